# Initial kernel scaffold; baseline (speedup 1.0000x reference)
import jax
import jax.numpy as jnp
from jax.experimental import pallas as pl


def kernel(input, adj, W):
    raise NotImplementedError("write your pallas kernel here")


# speedup vs baseline: 60.6053x; 60.6053x over previous
import jax
import jax.numpy as jnp
from jax.experimental import pallas as pl


def _support_kernel(x_ref, w_ref, out_ref):
    out_ref[...] = jax.lax.dot_general(
        x_ref[...], w_ref[...], (((1,), (0,)), ((), ())),
        preferred_element_type=jnp.float32,
        precision=jax.lax.Precision.HIGHEST,
    )


def _agg_kernel(adj_ref, s_ref, out_ref):
    acc = jax.lax.dot_general(
        adj_ref[...], s_ref[...], (((1,), (0,)), ((), ())),
        preferred_element_type=jnp.float32,
        precision=jax.lax.Precision.HIGHEST,
    )
    out_ref[...] = jnp.maximum(acc, 0.0)


def kernel(input, adj, W):
    n, f_in = input.shape
    f_out = W.shape[1]
    x = input.astype(jnp.float32)
    adj32 = adj.astype(jnp.float32)
    w = W.astype(jnp.float32)

    _i32 = lambda v: jax.lax.convert_element_type(v, jnp.int32)
    support = pl.pallas_call(
        _support_kernel,
        out_shape=jax.ShapeDtypeStruct((n, f_out), jnp.float32),
        grid=(1,),
        in_specs=[
            pl.BlockSpec((n, f_in), lambda i: (_i32(0), _i32(0))),
            pl.BlockSpec((f_in, f_out), lambda i: (_i32(0), _i32(0))),
        ],
        out_specs=pl.BlockSpec((n, f_out), lambda i: (_i32(0), _i32(0))),
    )(x, w)

    bm = 400
    out = pl.pallas_call(
        _agg_kernel,
        out_shape=jax.ShapeDtypeStruct((n, f_out), jnp.float32),
        grid=(n // bm,),
        in_specs=[
            pl.BlockSpec((bm, n), lambda i: (_i32(i), _i32(0))),
            pl.BlockSpec((n, f_out), lambda i: (_i32(0), _i32(0))),
        ],
        out_specs=pl.BlockSpec((bm, f_out), lambda i: (_i32(i), _i32(0))),
    )(adj32, support)

    return out.astype(jnp.float64)


# speedup vs baseline: 128.4965x; 2.1202x over previous
import jax
import jax.numpy as jnp
from jax.experimental import pallas as pl


def _support_kernel(x_ref, w_ref, out_ref):
    out_ref[...] = jax.lax.dot_general(
        x_ref[...], w_ref[...], (((1,), (0,)), ((), ())),
        preferred_element_type=jnp.float32,
        precision=jax.lax.Precision.HIGHEST,
    )


def _agg_kernel(adj_ref, s_ref, out_ref):
    acc = jax.lax.dot_general(
        adj_ref[...], s_ref[...], (((1,), (0,)), ((), ())),
        preferred_element_type=jnp.float32,
        precision=jax.lax.Precision.DEFAULT,
    )
    out_ref[...] = jnp.maximum(acc, 0.0)


def kernel(input, adj, W):
    n, f_in = input.shape
    f_out = W.shape[1]
    x = input.astype(jnp.float32)
    adj32 = adj.astype(jnp.float32)
    w = W.astype(jnp.float32)

    _i32 = lambda v: jax.lax.convert_element_type(v, jnp.int32)
    support = pl.pallas_call(
        _support_kernel,
        out_shape=jax.ShapeDtypeStruct((n, f_out), jnp.float32),
        grid=(1,),
        in_specs=[
            pl.BlockSpec((n, f_in), lambda i: (_i32(0), _i32(0))),
            pl.BlockSpec((f_in, f_out), lambda i: (_i32(0), _i32(0))),
        ],
        out_specs=pl.BlockSpec((n, f_out), lambda i: (_i32(0), _i32(0))),
    )(x, w)

    bm = 400
    out = pl.pallas_call(
        _agg_kernel,
        out_shape=jax.ShapeDtypeStruct((n, f_out), jnp.float32),
        grid=(n // bm,),
        in_specs=[
            pl.BlockSpec((bm, n), lambda i: (_i32(i), _i32(0))),
            pl.BlockSpec((n, f_out), lambda i: (_i32(0), _i32(0))),
        ],
        out_specs=pl.BlockSpec((bm, f_out), lambda i: (_i32(i), _i32(0))),
    )(adj32, support)

    return out.astype(jnp.float64)


# speedup vs baseline: 129.1142x; 1.0048x over previous
import jax
import jax.numpy as jnp
from jax.experimental import pallas as pl


def _support_kernel(x_ref, w_ref, out_ref):
    out_ref[...] = jax.lax.dot_general(
        x_ref[...], w_ref[...], (((1,), (0,)), ((), ())),
        preferred_element_type=jnp.float32,
        precision=jax.lax.Precision.HIGHEST,
    )


def _agg_kernel(adj_ref, s_ref, out_ref):
    acc = jax.lax.dot_general(
        adj_ref[...], s_ref[...], (((1,), (0,)), ((), ())),
        preferred_element_type=jnp.float32,
        precision=jax.lax.Precision.DEFAULT,
    )
    out_ref[...] = jnp.maximum(acc, 0.0)


def kernel(input, adj, W):
    n, f_in = input.shape
    f_out = W.shape[1]
    x = input.astype(jnp.float32)
    adj32 = adj.astype(jnp.float32)
    w = W.astype(jnp.float32)

    _i32 = lambda v: jax.lax.convert_element_type(v, jnp.int32)
    support = pl.pallas_call(
        _support_kernel,
        out_shape=jax.ShapeDtypeStruct((n, f_out), jnp.float32),
        grid=(1,),
        in_specs=[
            pl.BlockSpec((n, f_in), lambda i: (_i32(0), _i32(0))),
            pl.BlockSpec((f_in, f_out), lambda i: (_i32(0), _i32(0))),
        ],
        out_specs=pl.BlockSpec((n, f_out), lambda i: (_i32(0), _i32(0))),
    )(x, w)

    bm = 200
    out = pl.pallas_call(
        _agg_kernel,
        out_shape=jax.ShapeDtypeStruct((n, f_out), jnp.float32),
        grid=(n // bm,),
        in_specs=[
            pl.BlockSpec((bm, n), lambda i: (_i32(i), _i32(0))),
            pl.BlockSpec((n, f_out), lambda i: (_i32(0), _i32(0))),
        ],
        out_specs=pl.BlockSpec((bm, f_out), lambda i: (_i32(i), _i32(0))),
    )(adj32, support)

    return out.astype(jnp.float64)


# speedup vs baseline: 132.6435x; 1.0273x over previous
import jax
import jax.numpy as jnp
from jax.experimental import pallas as pl
from jax.experimental.pallas import tpu as pltpu


def _gcn_kernel(x_ref, w_ref, adj_ref, out_ref, s_ref):
    @pl.when(pl.program_id(0) == 0)
    def _():
        s_ref[...] = jax.lax.dot_general(
            x_ref[...], w_ref[...], (((1,), (0,)), ((), ())),
            preferred_element_type=jnp.float32,
            precision=jax.lax.Precision.HIGHEST,
        )

    acc = jax.lax.dot_general(
        adj_ref[...], s_ref[...], (((1,), (0,)), ((), ())),
        preferred_element_type=jnp.float32,
        precision=jax.lax.Precision.DEFAULT,
    )
    out_ref[...] = jnp.maximum(acc, 0.0)


def kernel(input, adj, W):
    n, f_in = input.shape
    f_out = W.shape[1]
    x = input.astype(jnp.float32)
    adj32 = adj.astype(jnp.float32)
    w = W.astype(jnp.float32)

    _i32 = lambda v: jax.lax.convert_element_type(v, jnp.int32)
    bm = 200
    out = pl.pallas_call(
        _gcn_kernel,
        out_shape=jax.ShapeDtypeStruct((n, f_out), jnp.float32),
        grid=(n // bm,),
        in_specs=[
            pl.BlockSpec((n, f_in), lambda i: (_i32(0), _i32(0))),
            pl.BlockSpec((f_in, f_out), lambda i: (_i32(0), _i32(0))),
            pl.BlockSpec((bm, n), lambda i: (_i32(i), _i32(0))),
        ],
        out_specs=pl.BlockSpec((bm, f_out), lambda i: (_i32(i), _i32(0))),
        scratch_shapes=[pltpu.VMEM((n, f_out), jnp.float32)],
    )(x, w, adj32)

    return out.astype(jnp.float64)
